# baseline (speedup 1.0000x reference)
import functools

import jax
import jax.numpy as jnp
from jax import lax
from jax.experimental import pallas as pl
from jax.experimental.pallas import tpu as pltpu
from jax.experimental.pallas import tpu_sc as plsc

VOCAB = 1000000
BATCH = 16384
HIST = 50
D = 64

NC = 2
NS = 16
NW = NC * NS

ROWS_PER_W = BATCH // NW
CHUNK_ROWS = 4
IDX_PER_CHUNK = CHUNK_ROWS * HIST
NCHUNK = ROWS_PER_W // CHUNK_ROWS
LANES = 16
DSUB = D // LANES
SHIFT_VECS = (IDX_PER_CHUNK + LANES - 1) // LANES


def _repack_body(lo_ref, hi_ref, o_ref):
    o_ref[...] = jnp.concatenate([lo_ref[...], hi_ref[...]], axis=1)


def _repack(table):
    rows_blk = 10000
    grid = (VOCAB // 2) // rows_blk
    nblk = grid
    return pl.pallas_call(
        _repack_body,
        grid=(grid,),
        in_specs=[
            pl.BlockSpec((rows_blk, D), lambda i: (i, 0)),
            pl.BlockSpec((rows_blk, D), lambda i: (i + nblk, 0)),
        ],
        out_specs=pl.BlockSpec((rows_blk, 2 * D), lambda i: (i, 0)),
        out_shape=jax.ShapeDtypeStruct((VOCAB // 2, 2 * D), jnp.float32),
    )(table, table)


def _sc_body(actions_hbm, t2_hbm, out_hbm,
             idx_v, sidx0, sidx1, buf0, buf1, out_v, sem0, sem1):
    wid = lax.axis_index("s") * NC + lax.axis_index("c")
    base_idx = wid * ROWS_PER_W * HIST

    pltpu.sync_copy(actions_hbm.at[pl.ds(pl.multiple_of(base_idx, 8),
                                         ROWS_PER_W * HIST)],
                    idx_v.at[pl.ds(0, ROWS_PER_W * HIST)])

    sidxs = (sidx0, sidx1)
    bufs = (buf0, buf1)
    sems = (sem0, sem1)

    def _start_gather(chunk, sidx, buf, sem):
        for k in range(SHIFT_VECS):
            off = chunk * IDX_PER_CHUNK + k * LANES
            a = idx_v[pl.ds(off, LANES)]
            m = lax.shift_right_arithmetic(jnp.int32(VOCAB // 2 - 1) - a, 31)
            sidx[pl.ds(k * LANES, LANES)] = a + m * jnp.int32(VOCAB // 2)
        return pltpu.async_copy(
            t2_hbm.at[sidx.at[pl.ds(0, IDX_PER_CHUNK)]], buf, sem)

    _start_gather(0, sidx0, buf0, sem0)
    _start_gather(1, sidx1, buf1, sem1)

    def _reduce_chunk(chunk, buf):
        def _row(r, carry):
            e0 = (chunk * CHUNK_ROWS + r) * HIST
            iv = [idx_v[pl.ds(e0, LANES)],
                  idx_v[pl.ds(e0 + 16, LANES)],
                  idx_v[pl.ds(e0 + 32, LANES)],
                  idx_v[pl.ds(e0 + 34, LANES)]]

            def col(i):
                a = iv[i // 16][i % 16] if i < 48 else iv[3][i - 34]
                return lax.shift_right_arithmetic(
                    jnp.int32(VOCAB // 2 - 1) - a, 31) & jnp.int32(D)

            acc = [buf[r * HIST, pl.ds(col(0) + j * LANES, LANES)]
                   for j in range(DSUB)]
            for i in range(1, HIST):
                ci = col(i)
                for j in range(DSUB):
                    acc[j] = acc[j] + buf[r * HIST + i,
                                          pl.ds(ci + j * LANES, LANES)]
            obase = (chunk * CHUNK_ROWS + r) * D
            for j in range(DSUB):
                out_v[pl.ds(obase + j * LANES, LANES)] = acc[j]
            return carry
        lax.fori_loop(0, CHUNK_ROWS, _row, 0, unroll=False)

    def _step(i, carry):
        for p in range(2):
            chunk = 2 * i + p
            pltpu.make_async_copy(
                t2_hbm.at[sidxs[p].at[pl.ds(0, IDX_PER_CHUNK)]],
                bufs[p], sems[p]).wait()
            _reduce_chunk(chunk, bufs[p])

            @pl.when(i < NCHUNK // 2 - 1)
            def _start_next(p=p, chunk=chunk):
                _start_gather(chunk + 2, sidxs[p], bufs[p], sems[p])
        return carry

    lax.fori_loop(0, NCHUNK // 2, _step, 0, unroll=False)

    pltpu.sync_copy(out_v,
                    out_hbm.at[pl.ds(pl.multiple_of(wid * ROWS_PER_W * D, 8),
                                     ROWS_PER_W * D)])


def _sc_gather_pool(actions_flat, t2):
    mesh = plsc.VectorSubcoreMesh(core_axis_name="c", subcore_axis_name="s",
                                  num_cores=NC, num_subcores=NS)
    fn = pl.kernel(
        _sc_body,
        out_type=jax.ShapeDtypeStruct((BATCH * D,), jnp.float32),
        mesh=mesh,
        scratch_types=[
            pltpu.VMEM((ROWS_PER_W * HIST + LANES,), jnp.int32),
            pltpu.VMEM((SHIFT_VECS * LANES,), jnp.int32),
            pltpu.VMEM((SHIFT_VECS * LANES,), jnp.int32),
            pltpu.VMEM((IDX_PER_CHUNK, 2 * D), jnp.float32),
            pltpu.VMEM((IDX_PER_CHUNK, 2 * D), jnp.float32),
            pltpu.VMEM((ROWS_PER_W * D,), jnp.float32),
            pltpu.SemaphoreType.DMA,
            pltpu.SemaphoreType.DMA,
        ],
    )
    return fn(actions_flat, t2)


def _tc_project_body(x_ref, w_ref, b_ref, o_ref):
    x = x_ref[...] * (1.0 / HIST)
    o_ref[...] = lax.dot_general(
        x, w_ref[...], (((1,), (1,)), ((), ())),
        preferred_element_type=jnp.float32) + b_ref[...]


def _tc_project(pooled, w, b2):
    bm = 1024
    return pl.pallas_call(
        _tc_project_body,
        grid=(BATCH // bm,),
        in_specs=[
            pl.BlockSpec((bm, D), lambda i: (i, 0)),
            pl.BlockSpec((D, D), lambda i: (0, 0)),
            pl.BlockSpec((1, D), lambda i: (0, 0)),
        ],
        out_specs=pl.BlockSpec((bm, D), lambda i: (i, 0)),
        out_shape=jax.ShapeDtypeStruct((BATCH, D), jnp.float32),
    )(pooled, w, b2)


def kernel(actions, emb_table, W, b):
    actions_flat = actions.reshape(-1).astype(jnp.int32)
    t2 = _repack(emb_table)
    pooled = _sc_gather_pool(actions_flat, t2).reshape(BATCH, D)
    return _tc_project(pooled, W, b.reshape(1, D))
